# baseline (speedup 1.0000x reference)
import jax
import jax.numpy as jnp
from jax import lax
from jax.experimental import pallas as pl
from jax.experimental.pallas import tpu as pltpu
from jax.experimental.pallas import tpu_sc as plsc
import functools

B = 16384
N_FIELDS = 26
VOCAB = 100000
VSTRIDE = 100000
N_NUM = 13
NF0 = 13
NF1 = N_FIELDS - NF0

NC = 2
NS = 16
L = 16
NW = NC * NS
BPW = B // NW
NV = BPW // L


def _fm_body(xcT_hbm, xnT_hbm, table0_hbm, table1_hbm, wb_hbm, bb_hbm, out_hbm,
             idx_v, xn_v, vals_v, wb_v, bb_v, acc_v, out_v,
             s0, s1, sn, g0, g1):
    wid = lax.axis_index("s") * NC + lax.axis_index("c")
    base = wid * BPW

    cp0 = [
        pltpu.async_copy(xcT_hbm.at[pl.ds(f * B + base, BPW)],
                         idx_v.at[pl.ds(f * BPW, BPW)], s0)
        for f in range(NF0)
    ]
    cp1 = [
        pltpu.async_copy(xcT_hbm.at[pl.ds(f * B + base, BPW)],
                         idx_v.at[pl.ds(f * BPW, BPW)], s1)
        for f in range(NF0, N_FIELDS)
    ]
    cpn = [
        pltpu.async_copy(xnT_hbm.at[pl.ds(f * B + base, BPW)],
                         xn_v.at[pl.ds(f * BPW, BPW)], sn)
        for f in range(N_NUM)
    ]
    pltpu.sync_copy(wb_hbm, wb_v)
    pltpu.sync_copy(bb_hbm, bb_v)

    for c in cp0:
        c.wait()

    def off0(j, _):
        for f in range(1, NF0):
            s = pl.ds(f * BPW + j * L, L)
            idx_v[s] = idx_v[s] + f * VSTRIDE
        return 0

    lax.fori_loop(0, NV, off0, 0)
    gcp0 = pltpu.async_copy(
        table0_hbm.at[idx_v.at[pl.ds(0, NF0 * BPW)]],
        vals_v.at[pl.ds(0, NF0 * BPW)], g0)

    for c in cp1:
        c.wait()

    def off1(j, _):
        for f in range(NF0, N_FIELDS):
            s = pl.ds(f * BPW + j * L, L)
            idx_v[s] = idx_v[s] + (f - NF0) * VSTRIDE
        return 0

    lax.fori_loop(0, NV, off1, 0)
    gcp1 = pltpu.async_copy(
        table1_hbm.at[idx_v.at[pl.ds(NF0 * BPW, NF1 * BPW)]],
        vals_v.at[pl.ds(NF0 * BPW, NF1 * BPW)], g1)

    for c in cpn:
        c.wait()
    gcp0.wait()

    def red0(j, _):
        acc = bb_v[:]
        for f in range(NF0):
            acc = acc + vals_v[pl.ds(f * BPW + j * L, L)]
        for f in range(N_NUM):
            acc = acc + xn_v[pl.ds(f * BPW + j * L, L)] * wb_v[pl.ds(f * L, L)]
        acc_v[pl.ds(j * L, L)] = acc
        return 0

    lax.fori_loop(0, NV, red0, 0)
    gcp1.wait()

    def red1(j, _):
        acc = acc_v[pl.ds(j * L, L)]
        for f in range(NF0, N_FIELDS):
            acc = acc + vals_v[pl.ds(f * BPW + j * L, L)]
        out_v[pl.ds(j * L, L)] = 1.0 / (1.0 + jnp.exp(-acc))
        return 0

    lax.fori_loop(0, NV, red1, 0)

    pltpu.sync_copy(out_v, out_hbm.at[pl.ds(base, BPW)])


@functools.partial(
    pl.kernel,
    out_type=jax.ShapeDtypeStruct((B,), jnp.float32),
    mesh=plsc.VectorSubcoreMesh(core_axis_name="c", subcore_axis_name="s"),
    compiler_params=pltpu.CompilerParams(needs_layout_passes=False),
    scratch_types=[
        pltpu.VMEM((N_FIELDS * BPW,), jnp.int32),
        pltpu.VMEM((N_NUM * BPW,), jnp.float32),
        pltpu.VMEM((N_FIELDS * BPW,), jnp.float32),
        pltpu.VMEM((N_NUM * L,), jnp.float32),
        pltpu.VMEM((L,), jnp.float32),
        pltpu.VMEM((BPW,), jnp.float32),
        pltpu.VMEM((BPW,), jnp.float32),
        pltpu.SemaphoreType.DMA,
        pltpu.SemaphoreType.DMA,
        pltpu.SemaphoreType.DMA,
        pltpu.SemaphoreType.DMA,
        pltpu.SemaphoreType.DMA,
    ],
)
def _fm_sc(*args):
    _fm_body(*args)


def kernel(Xc, Xn, emb_tables, W, b):
    rows = [emb_tables[f, :, 0] for f in range(N_FIELDS)]
    t0 = jnp.concatenate(rows[:NF0])
    t1 = jnp.concatenate(rows[NF0:])
    xcT = Xc.T.reshape(-1)
    xnT = Xn.T.reshape(-1)
    wb = jnp.broadcast_to(W[0][:, None], (N_NUM, L)).reshape(-1)
    bb = jnp.broadcast_to(b, (L,))
    out = _fm_sc(xcT, xnT, t0, t1, wb, bb)
    return out[:, None]


# speedup vs baseline: 4.1365x; 4.1365x over previous
import jax
import jax.numpy as jnp
from jax import lax
from jax.experimental import pallas as pl
from jax.experimental.pallas import tpu as pltpu
from jax.experimental.pallas import tpu_sc as plsc
import functools

B = 16384
N_FIELDS = 26
VOCAB = 100000
VSTRIDE = 100096
N_NUM = 13
NF0 = 13
NF1 = N_FIELDS - NF0

NC = 2
NS = 16
L = 16
NW = NC * NS
BPW = B // NW
NV = BPW // L


def _fm_body(xcT_hbm, xnT_hbm, table_hbm, wb_hbm, bb_hbm, out_hbm,
             idx_v, xn_v, vals_v, wb_v, bb_v, acc_v, out_v,
             s0, s1, sn, g0, g1):
    wid = lax.axis_index("s") * NC + lax.axis_index("c")
    base = wid * BPW

    cp0 = [
        pltpu.async_copy(xcT_hbm.at[pl.ds(f * B + base, BPW)],
                         idx_v.at[pl.ds(f * BPW, BPW)], s0)
        for f in range(NF0)
    ]
    cp1 = [
        pltpu.async_copy(xcT_hbm.at[pl.ds(f * B + base, BPW)],
                         idx_v.at[pl.ds(f * BPW, BPW)], s1)
        for f in range(NF0, N_FIELDS)
    ]
    cpn = [
        pltpu.async_copy(xnT_hbm.at[pl.ds(f * B + base, BPW)],
                         xn_v.at[pl.ds(f * BPW, BPW)], sn)
        for f in range(N_NUM)
    ]
    pltpu.sync_copy(wb_hbm, wb_v)
    pltpu.sync_copy(bb_hbm, bb_v)

    for c in cp0:
        c.wait()

    gcp0 = [pltpu.async_copy(
        table_hbm.at[f, 0].at[idx_v.at[pl.ds(f * BPW, BPW)]],
        vals_v.at[pl.ds(f * BPW, BPW)], g0) for f in range(NF0)]

    for c in cp1:
        c.wait()

    gcp1 = [pltpu.async_copy(
        table_hbm.at[f, 0].at[idx_v.at[pl.ds(f * BPW, BPW)]],
        vals_v.at[pl.ds(f * BPW, BPW)], g1) for f in range(NF0, N_FIELDS)]

    for c in cpn:
        c.wait()
    for c in gcp0:
        c.wait()

    def red0(j, _):
        acc = bb_v[:]
        for f in range(NF0):
            acc = acc + vals_v[pl.ds(f * BPW + j * L, L)]
        for f in range(N_NUM):
            acc = acc + xn_v[pl.ds(f * BPW + j * L, L)] * wb_v[pl.ds(f * L, L)]
        acc_v[pl.ds(j * L, L)] = acc
        return 0

    lax.fori_loop(0, NV, red0, 0)
    for c in gcp1:
        c.wait()

    def red1(j, _):
        acc = acc_v[pl.ds(j * L, L)]
        for f in range(NF0, N_FIELDS):
            acc = acc + vals_v[pl.ds(f * BPW + j * L, L)]
        out_v[pl.ds(j * L, L)] = 1.0 / (1.0 + jnp.exp(-acc))
        return 0

    lax.fori_loop(0, NV, red1, 0)

    pltpu.sync_copy(out_v, out_hbm.at[pl.ds(base, BPW)])


@functools.partial(
    pl.kernel,
    out_type=jax.ShapeDtypeStruct((B,), jnp.float32),
    mesh=plsc.VectorSubcoreMesh(core_axis_name="c", subcore_axis_name="s"),
    compiler_params=pltpu.CompilerParams(needs_layout_passes=False),
    scratch_types=[
        pltpu.VMEM((N_FIELDS * BPW,), jnp.int32),
        pltpu.VMEM((N_NUM * BPW,), jnp.float32),
        pltpu.VMEM((N_FIELDS * BPW,), jnp.float32),
        pltpu.VMEM((N_NUM * L,), jnp.float32),
        pltpu.VMEM((L,), jnp.float32),
        pltpu.VMEM((BPW,), jnp.float32),
        pltpu.VMEM((BPW,), jnp.float32),
        pltpu.SemaphoreType.DMA,
        pltpu.SemaphoreType.DMA,
        pltpu.SemaphoreType.DMA,
        pltpu.SemaphoreType.DMA,
        pltpu.SemaphoreType.DMA,
    ],
)
def _fm_sc(*args):
    _fm_body(*args)


def kernel(Xc, Xn, emb_tables, W, b):
    table_t = jnp.transpose(emb_tables, (0, 2, 1))
    xcT = Xc.T.reshape(-1)
    xnT = Xn.T.reshape(-1)
    wb = jnp.broadcast_to(W[0][:, None], (N_NUM, L)).reshape(-1)
    bb = jnp.broadcast_to(b, (L,))
    out = _fm_sc(xcT, xnT, table_t, wb, bb)
    return out[:, None]
